# Initial kernel scaffold; baseline (speedup 1.0000x reference)
import jax
import jax.numpy as jnp
from jax.experimental import pallas as pl


def kernel(inputs, table):
    raise NotImplementedError("write your pallas kernel here")


# speedup vs baseline: 4.0968x; 4.0968x over previous
import functools
import math

import jax
import jax.numpy as jnp
from jax import lax
from jax.experimental import pallas as pl
from jax.experimental.pallas import tpu as pltpu
from jax.experimental.pallas import tpu_sc as plsc

_VOCAB = 1000
_EMB = 128
_BATCH = 4096
_SEQ = 200
_SCALE = float(_EMB) ** 0.5

_NC = 2
_NS = 16
_NW = _NC * _NS

_ROWS = _BATCH * _SEQ
_ROWS_PER_W = _ROWS // _NW
_CHUNK = _SEQ
_NCHUNK = _ROWS_PER_W // _CHUNK
_SUB = 40
_NSUB = _CHUNK // _SUB


def _sinusoid_signal():
    position = jnp.arange(_SEQ, dtype=jnp.float32)
    num_ts = _EMB // 2
    inc = math.log(10000.0) / (num_ts - 1)
    inv_ts = jnp.exp(jnp.arange(num_ts, dtype=jnp.float32) * -inc)
    scaled = position[:, None] * inv_ts[None, :]
    return jnp.concatenate([jnp.sin(scaled), jnp.cos(scaled)], axis=1)


@functools.partial(
    pl.kernel,
    out_type=jax.ShapeDtypeStruct((_ROWS, _EMB), jnp.float32),
    mesh=plsc.VectorSubcoreMesh(core_axis_name="c", subcore_axis_name="s"),
    scratch_types=[
        pltpu.VMEM((_CHUNK,), jnp.int32),
        pltpu.VMEM((_CHUNK, _EMB), jnp.float32),
        pltpu.VMEM((_SEQ, _EMB), jnp.float32),
        pltpu.SemaphoreType.DMA,
    ],
)
def _embed_kernel(idx_hbm, table_hbm, sig_hbm, out_hbm, idx_v, rows_v, sig_v, sem):
    wid = lax.axis_index("s") * _NC + lax.axis_index("c")
    row_base_w = wid * _ROWS_PER_W

    pltpu.sync_copy(sig_hbm, sig_v)

    def chunk_body(q, carry):
        row_base = row_base_w + q * _CHUNK
        pltpu.sync_copy(idx_hbm.at[pl.ds(row_base, _CHUNK)], idx_v)
        for j in range(_NSUB):
            pltpu.async_copy(
                table_hbm.at[idx_v.at[pl.ds(j * _SUB, _SUB)]],
                rows_v.at[pl.ds(j * _SUB, _SUB)],
                sem,
            )
        pltpu.make_async_copy(
            table_hbm.at[pl.ds(0, _CHUNK)], rows_v, sem
        ).wait()

        def row_body(s, c2):
            for c in range(_EMB // 16):
                sl = pl.ds(c * 16, 16)
                rows_v[s, sl] = rows_v[s, sl] * _SCALE + sig_v[s, sl]
            return c2

        lax.fori_loop(0, _CHUNK, row_body, 0, unroll=False)
        pltpu.sync_copy(rows_v, out_hbm.at[pl.ds(row_base, _CHUNK)])
        return carry

    lax.fori_loop(0, _NCHUNK, chunk_body, 0, unroll=False)


def kernel(inputs, table):
    idx = inputs.astype(jnp.int32).reshape(_ROWS)
    sig = _sinusoid_signal()
    out = _embed_kernel(idx, table, sig)
    return out.reshape(_BATCH, _SEQ, _EMB)


# speedup vs baseline: 4.8563x; 1.1854x over previous
import functools
import math

import jax
import jax.numpy as jnp
from jax import lax
from jax.experimental import pallas as pl
from jax.experimental.pallas import tpu as pltpu
from jax.experimental.pallas import tpu_sc as plsc

_VOCAB = 1000
_VOCAB_PAD = 1024
_EMB = 128
_BATCH = 4096
_SEQ = 200
_SCALE = float(_EMB) ** 0.5

_NC = 2
_NS = 16
_NW = _NC * _NS

_ROWS = _BATCH * _SEQ
_ROWS_PER_W = _ROWS // _NW
_CHUNK = _SEQ
_NCHUNK = _ROWS_PER_W // _CHUNK
_SUB = 40
_NSUB = _CHUNK // _SUB
_TROWS = _VOCAB_PAD // _NS


def _sinusoid_signal():
    position = jnp.arange(_SEQ, dtype=jnp.float32)
    num_ts = _EMB // 2
    inc = math.log(10000.0) / (num_ts - 1)
    inv_ts = jnp.exp(jnp.arange(num_ts, dtype=jnp.float32) * -inc)
    scaled = position[:, None] * inv_ts[None, :]
    return jnp.concatenate([jnp.sin(scaled), jnp.cos(scaled)], axis=1)


@functools.partial(
    pl.kernel,
    out_type=jax.ShapeDtypeStruct((_ROWS, _EMB), jnp.float32),
    mesh=plsc.VectorSubcoreMesh(core_axis_name="c", subcore_axis_name="s"),
    scratch_types=[
        pltpu.VMEM((_CHUNK,), jnp.int32),
        pltpu.VMEM((_CHUNK, _EMB), jnp.float32),
        pltpu.VMEM((_SEQ, _EMB), jnp.float32),
        pltpu.VMEM_SHARED((_VOCAB_PAD, _EMB), jnp.float32),
        pltpu.SemaphoreType.DMA,
    ],
)
def _embed_kernel(idx_hbm, table_hbm, sig_hbm, out_hbm,
                  idx_v, rows_v, sig_v, table_sp, sem):
    sid = lax.axis_index("s")
    wid = sid * _NC + lax.axis_index("c")
    row_base_w = wid * _ROWS_PER_W

    trow = sid * _TROWS
    pltpu.sync_copy(table_hbm.at[pl.ds(trow, _TROWS)], rows_v.at[pl.ds(0, _TROWS)])

    def scale_body(r, c2):
        for c in range(_EMB // 16):
            sl = pl.ds(c * 16, 16)
            rows_v[r, sl] = rows_v[r, sl] * _SCALE
        return c2

    lax.fori_loop(0, _TROWS, scale_body, 0, unroll=False)
    pltpu.sync_copy(rows_v.at[pl.ds(0, _TROWS)], table_sp.at[pl.ds(trow, _TROWS)])
    pltpu.sync_copy(sig_hbm, sig_v)
    plsc.subcore_barrier()

    def chunk_body(q, carry):
        row_base = row_base_w + q * _CHUNK
        pltpu.sync_copy(idx_hbm.at[pl.ds(row_base, _CHUNK)], idx_v)
        for j in range(_NSUB):
            pltpu.async_copy(
                table_sp.at[idx_v.at[pl.ds(j * _SUB, _SUB)]],
                rows_v.at[pl.ds(j * _SUB, _SUB)],
                sem,
            )
        pltpu.make_async_copy(
            table_hbm.at[pl.ds(0, _CHUNK)], rows_v, sem
        ).wait()

        def row_body(s, c2):
            for c in range(_EMB // 16):
                sl = pl.ds(c * 16, 16)
                plsc.addupdate(rows_v.at[s, sl], sig_v[s, sl])
            return c2

        lax.fori_loop(0, _CHUNK, row_body, 0, unroll=False)
        pltpu.sync_copy(rows_v, out_hbm.at[pl.ds(row_base, _CHUNK)])
        return carry

    lax.fori_loop(0, _NCHUNK, chunk_body, 0, unroll=False)


def kernel(inputs, table):
    idx = inputs.astype(jnp.int32).reshape(_ROWS)
    table_p = jnp.pad(table, ((0, _VOCAB_PAD - _VOCAB), (0, 0)))
    sig = _sinusoid_signal()
    out = _embed_kernel(idx, table_p, sig)
    return out.reshape(_BATCH, _SEQ, _EMB)


# speedup vs baseline: 6.1881x; 1.2742x over previous
import functools
import math

import jax
import jax.numpy as jnp
from jax import lax
from jax.experimental import pallas as pl
from jax.experimental.pallas import tpu as pltpu
from jax.experimental.pallas import tpu_sc as plsc

_VOCAB = 1000
_VOCAB_PAD = 1024
_EMB = 128
_BATCH = 4096
_SEQ = 200
_SCALE = float(_EMB) ** 0.5

_NC = 2
_NS = 16
_NW = _NC * _NS

_ROWS = _BATCH * _SEQ
_ROWS_PER_W = _ROWS // _NW
_CHUNK = _SEQ
_NCHUNK = _ROWS_PER_W // _CHUNK
_SUB = 40
_NSUB = _CHUNK // _SUB
_TROWS = _VOCAB_PAD // _NS


def _sinusoid_signal():
    position = jnp.arange(_SEQ, dtype=jnp.float32)
    num_ts = _EMB // 2
    inc = math.log(10000.0) / (num_ts - 1)
    inv_ts = jnp.exp(jnp.arange(num_ts, dtype=jnp.float32) * -inc)
    scaled = position[:, None] * inv_ts[None, :]
    return jnp.concatenate([jnp.sin(scaled), jnp.cos(scaled)], axis=1)


@functools.partial(
    pl.kernel,
    out_type=jax.ShapeDtypeStruct((_ROWS, _EMB), jnp.float32),
    mesh=plsc.VectorSubcoreMesh(core_axis_name="c", subcore_axis_name="s"),
    scratch_types=[
        pltpu.VMEM((_CHUNK,), jnp.int32),
        pltpu.VMEM((_CHUNK,), jnp.int32),
        pltpu.VMEM((_CHUNK, _EMB), jnp.float32),
        pltpu.VMEM((_CHUNK, _EMB), jnp.float32),
        pltpu.VMEM((_SEQ, _EMB), jnp.float32),
        pltpu.VMEM_SHARED((_VOCAB_PAD, _EMB), jnp.float32),
        pltpu.SemaphoreType.DMA,
        pltpu.SemaphoreType.DMA,
        pltpu.SemaphoreType.DMA,
        pltpu.SemaphoreType.DMA,
    ],
)
def _embed_kernel(idx_hbm, table_hbm, sig_hbm, out_hbm,
                  idx_v0, idx_v1, rows_v0, rows_v1, sig_v, table_sp,
                  sem_g0, sem_g1, sem_o0, sem_o1):
    sid = lax.axis_index("s")
    wid = sid * _NC + lax.axis_index("c")
    row_base_w = wid * _ROWS_PER_W
    idx_v = (idx_v0, idx_v1)
    rows_v = (rows_v0, rows_v1)
    sem_g = (sem_g0, sem_g1)
    sem_o = (sem_o0, sem_o1)

    trow = sid * _TROWS
    pltpu.sync_copy(table_hbm.at[pl.ds(trow, _TROWS)], rows_v0.at[pl.ds(0, _TROWS)])

    def scale_body(r, c2):
        for c in range(_EMB // 16):
            sl = pl.ds(c * 16, 16)
            rows_v0[r, sl] = rows_v0[r, sl] * _SCALE
        return c2

    lax.fori_loop(0, _TROWS, scale_body, 0, unroll=False)
    pltpu.sync_copy(rows_v0.at[pl.ds(0, _TROWS)], table_sp.at[pl.ds(trow, _TROWS)])
    pltpu.sync_copy(sig_hbm, sig_v)
    plsc.subcore_barrier()

    def start_gather(q, b):
        row_base = row_base_w + q * _CHUNK
        pltpu.sync_copy(idx_hbm.at[pl.ds(row_base, _CHUNK)], idx_v[b])
        for j in range(_NSUB):
            pltpu.async_copy(
                table_sp.at[idx_v[b].at[pl.ds(j * _SUB, _SUB)]],
                rows_v[b].at[pl.ds(j * _SUB, _SUB)],
                sem_g[b],
            )

    def wait_gather(b):
        pltpu.make_async_copy(
            table_hbm.at[pl.ds(0, _CHUNK)], rows_v[b], sem_g[b]
        ).wait()

    def wait_out(b):
        pltpu.make_async_copy(
            rows_v[b], out_hbm.at[pl.ds(0, _CHUNK)], sem_o[b]
        ).wait()

    start_gather(0, 0)

    def pair_body(g, carry):
        for b in range(2):
            q = 2 * g + b
            nb = 1 - b

            @pl.when(q + 1 < _NCHUNK)
            def _prefetch():
                @pl.when(q >= 1)
                def _():
                    wait_out(nb)
                start_gather(q + 1, nb)

            wait_gather(b)

            def row_body(s, c2):
                for c in range(_EMB // 16):
                    sl = pl.ds(c * 16, 16)
                    plsc.addupdate(rows_v[b].at[s, sl], sig_v[s, sl])
                return c2

            lax.fori_loop(0, _CHUNK, row_body, 0, unroll=False)
            row_base = row_base_w + q * _CHUNK
            pltpu.async_copy(rows_v[b], out_hbm.at[pl.ds(row_base, _CHUNK)],
                             sem_o[b])
        return carry

    lax.fori_loop(0, _NCHUNK // 2, pair_body, 0, unroll=False)
    wait_out(0)
    wait_out(1)


def kernel(inputs, table):
    idx = inputs.astype(jnp.int32).reshape(_ROWS)
    table_p = jnp.pad(table, ((0, _VOCAB_PAD - _VOCAB), (0, 0)))
    sig = _sinusoid_signal()
    out = _embed_kernel(idx, table_p, sig)
    return out.reshape(_BATCH, _SEQ, _EMB)


# speedup vs baseline: 7.4219x; 1.1994x over previous
import functools
import math

import jax
import jax.numpy as jnp
from jax import lax
from jax.experimental import pallas as pl
from jax.experimental.pallas import tpu as pltpu
from jax.experimental.pallas import tpu_sc as plsc

_VOCAB = 1000
_VOCAB_PAD = 1024
_EMB = 128
_BATCH = 4096
_SEQ = 200
_SCALE = float(_EMB) ** 0.5

_NC = 2
_NS = 16
_NW = _NC * _NS

_ROWS = _BATCH * _SEQ
_ROWS_PER_W = _ROWS // _NW
_CHUNK = _SEQ
_NCHUNK = _ROWS_PER_W // _CHUNK
_SUB = 40
_NSUB = _CHUNK // _SUB
_TROWS = _VOCAB_PAD // _NS


def _sinusoid_signal():
    position = jnp.arange(_SEQ, dtype=jnp.float32)
    num_ts = _EMB // 2
    inc = math.log(10000.0) / (num_ts - 1)
    inv_ts = jnp.exp(jnp.arange(num_ts, dtype=jnp.float32) * -inc)
    scaled = position[:, None] * inv_ts[None, :]
    return jnp.concatenate([jnp.sin(scaled), jnp.cos(scaled)], axis=1)


@functools.partial(
    pl.kernel,
    out_type=jax.ShapeDtypeStruct((_ROWS, _EMB), jnp.float32),
    mesh=plsc.VectorSubcoreMesh(core_axis_name="c", subcore_axis_name="s"),
    scratch_types=[
        pltpu.VMEM((_CHUNK,), jnp.int32),
        pltpu.VMEM((_CHUNK,), jnp.int32),
        pltpu.VMEM((_CHUNK, _EMB), jnp.float32),
        pltpu.VMEM((_CHUNK, _EMB), jnp.float32),
        pltpu.VMEM((_SEQ, _EMB), jnp.float32),
        pltpu.VMEM_SHARED((_VOCAB_PAD, _EMB), jnp.float32),
        pltpu.SemaphoreType.DMA,
        pltpu.SemaphoreType.DMA,
        pltpu.SemaphoreType.DMA,
        pltpu.SemaphoreType.DMA,
        pltpu.SemaphoreType.DMA,
        pltpu.SemaphoreType.DMA,
    ],
)
def _embed_kernel(idx_hbm, table_hbm, sig_hbm, out_hbm,
                  idx_v0, idx_v1, rows_v0, rows_v1, sig_v, table_sp,
                  sem_g0, sem_g1, sem_o0, sem_o1, sem_i0, sem_i1):
    sid = lax.axis_index("s")
    wid = sid * _NC + lax.axis_index("c")
    row_base_w = wid * _ROWS_PER_W
    idx_v = (idx_v0, idx_v1)
    rows_v = (rows_v0, rows_v1)
    sem_g = (sem_g0, sem_g1)
    sem_o = (sem_o0, sem_o1)
    sem_i = (sem_i0, sem_i1)

    trow = sid * _TROWS
    pltpu.sync_copy(table_hbm.at[pl.ds(trow, _TROWS)], rows_v0.at[pl.ds(0, _TROWS)])

    def scale_body(r, c2):
        for c in range(_EMB // 16):
            sl = pl.ds(c * 16, 16)
            rows_v0[r, sl] = rows_v0[r, sl] * _SCALE
        return c2

    lax.fori_loop(0, _TROWS, scale_body, 0, unroll=False)
    pltpu.sync_copy(rows_v0.at[pl.ds(0, _TROWS)], table_sp.at[pl.ds(trow, _TROWS)])
    pltpu.sync_copy(sig_hbm, sig_v)
    plsc.subcore_barrier()

    def start_idx(q, b):
        row_base = row_base_w + q * _CHUNK
        pltpu.async_copy(idx_hbm.at[pl.ds(row_base, _CHUNK)], idx_v[b],
                         sem_i[b])

    def wait_idx(b):
        pltpu.make_async_copy(
            idx_hbm.at[pl.ds(0, _CHUNK)], idx_v[b], sem_i[b]
        ).wait()

    def start_gather(b):
        for j in range(_NSUB):
            pltpu.async_copy(
                table_sp.at[idx_v[b].at[pl.ds(j * _SUB, _SUB)]],
                rows_v[b].at[pl.ds(j * _SUB, _SUB)],
                sem_g[b],
            )

    def wait_gather(b):
        pltpu.make_async_copy(
            table_hbm.at[pl.ds(0, _CHUNK)], rows_v[b], sem_g[b]
        ).wait()

    def wait_out(b):
        pltpu.make_async_copy(
            rows_v[b], out_hbm.at[pl.ds(0, _CHUNK)], sem_o[b]
        ).wait()

    pltpu.sync_copy(idx_hbm.at[pl.ds(row_base_w, _CHUNK)], idx_v0)
    start_gather(0)
    start_idx(1, 1)

    def pair_body(g, carry):
        for b in range(2):
            q = 2 * g + b
            nb = 1 - b

            @pl.when(q + 1 < _NCHUNK)
            def _prefetch():
                wait_idx(nb)

                @pl.when(q >= 1)
                def _():
                    wait_out(nb)
                start_gather(nb)

            wait_gather(b)

            @pl.when(q + 2 < _NCHUNK)
            def _prefetch_idx():
                start_idx(q + 2, b)

            def row_body(s, c2):
                for c in range(_EMB // 16):
                    sl = pl.ds(c * 16, 16)
                    plsc.addupdate(rows_v[b].at[s, sl], sig_v[s, sl])
                return c2

            lax.fori_loop(0, _CHUNK, row_body, 0, unroll=4)
            row_base = row_base_w + q * _CHUNK
            pltpu.async_copy(rows_v[b], out_hbm.at[pl.ds(row_base, _CHUNK)],
                             sem_o[b])
        return carry

    lax.fori_loop(0, _NCHUNK // 2, pair_body, 0, unroll=False)
    wait_out(0)
    wait_out(1)


def kernel(inputs, table):
    idx = inputs.astype(jnp.int32).reshape(_ROWS)
    table_p = jnp.pad(table, ((0, _VOCAB_PAD - _VOCAB), (0, 0)))
    sig = _sinusoid_signal()
    out = _embed_kernel(idx, table_p, sig)
    return out.reshape(_BATCH, _SEQ, _EMB)
